# baseline (speedup 1.0000x reference)
import functools

import jax
import jax.numpy as jnp
import numpy as np
from jax.experimental import pallas as pl

_TEMPERATURE = 0.02
_K = 10
_NEG = float(np.finfo(np.float32).min)


def _fused_body(row_ref, col_ref, unc_ref, ent_ref, *, k: int, max_ent: float):
    X = row_ref[...]
    Y = col_ref[...]
    blk = X.shape[0]

    tr = jnp.max(X, axis=1, keepdims=True)
    rowmax = tr
    for _ in range(k - 1):
        tr = jnp.max(jnp.where(X >= tr, _NEG, X), axis=1, keepdims=True)

    tc = jnp.max(Y, axis=0, keepdims=True)
    for _ in range(k - 1):
        tc = jnp.max(jnp.where(Y >= tc, _NEG, Y), axis=0, keepdims=True)

    inv_t = 1.0 / _TEMPERATURE
    sm = (X - rowmax) * inv_t
    e = jnp.exp(sm)
    Z = jnp.sum(e, axis=1, keepdims=True)
    S1 = jnp.sum(sm * e, axis=1, keepdims=True)
    ent = (jnp.log(Z) - S1 / Z)[:, 0] * (1.0 / max_ent)

    R = (X >= tr).astype(jnp.float32)
    C = (Y >= tc).astype(jnp.float32)
    P = jax.lax.dot(R, C, preferred_element_type=jnp.float32)
    ii = jax.lax.broadcasted_iota(jnp.int32, (blk, blk), 0)
    jj = jax.lax.broadcasted_iota(jnp.int32, (blk, blk), 1)
    matches = jnp.sum(jnp.where(ii == jj, P, 0.0), axis=1)

    ra = matches * (1.0 / k)
    unc_ref[...] = (1.0 - ra) * 0.5 + ent * 0.5
    ent_ref[...] = ent


def kernel(sim_matrix, pids):
    del pids
    B = sim_matrix.shape[0]
    blk = 512
    k = min(_K, B)
    max_ent = float(np.log(B + 1e-10))
    grid = B // blk
    unc, ent = pl.pallas_call(
        functools.partial(_fused_body, k=k, max_ent=max_ent),
        grid=(grid,),
        in_specs=[
            pl.BlockSpec((blk, B), lambda i: (i, 0)),
            pl.BlockSpec((B, blk), lambda i: (0, i)),
        ],
        out_specs=[
            pl.BlockSpec((blk,), lambda i: (i,)),
            pl.BlockSpec((blk,), lambda i: (i,)),
        ],
        out_shape=[
            jax.ShapeDtypeStruct((B,), jnp.float32),
            jax.ShapeDtypeStruct((B,), jnp.float32),
        ],
    )(sim_matrix, sim_matrix)
    return (unc, ent)
